# baseline (speedup 1.0000x reference)
import jax
import jax.numpy as jnp
from jax.experimental import pallas as pl
from jax.experimental.pallas import tpu as pltpu

CHANNEL_IN = 256
CHANNEL_OUT = 32768
GROUP = 8
BATCH = 128

COL_BLK = 4096


def _fused_kernel(x_ref, w_ref, o_ref):
    yt = jax.lax.dot_general(
        w_ref[...], x_ref[...], (((0,), (1,)), ((), ())),
        preferred_element_type=jnp.float32)
    y3 = yt.reshape(COL_BLK // GROUP, GROUP, BATCH)
    v = y3
    for k in (1, 2, 4):
        v = jnp.maximum(v, pltpu.roll(v, GROUP - k, 1))
    eqf = (y3 == v).astype(jnp.float32)
    o_ref[...] = eqf.reshape(COL_BLK, BATCH).T
    total = jnp.sum(eqf)

    @pl.when(total > float(COL_BLK // GROUP * BATCH))
    def _exact_tie_break():
        s = jax.lax.broadcasted_iota(
            jnp.int32, (COL_BLK // GROUP, GROUP, BATCH), 1).astype(jnp.float32)
        c = jnp.where(y3 == v, s, jnp.float32(GROUP))
        for k in (1, 2, 4):
            c = jnp.minimum(c, pltpu.roll(c, GROUP - k, 1))
        o_ref[...] = (s == c).astype(jnp.float32).reshape(COL_BLK, BATCH).T


def kernel(x, W):
    grid = (CHANNEL_OUT // COL_BLK,)
    return pl.pallas_call(
        _fused_kernel,
        grid=grid,
        in_specs=[
            pl.BlockSpec((BATCH, CHANNEL_IN), lambda j: (0, 0)),
            pl.BlockSpec((CHANNEL_IN, COL_BLK), lambda j: (0, j)),
        ],
        out_specs=pl.BlockSpec((BATCH, COL_BLK), lambda j: (0, j)),
        out_shape=jax.ShapeDtypeStruct((BATCH, CHANNEL_OUT), jnp.float32),
        compiler_params=pltpu.CompilerParams(
            dimension_semantics=("arbitrary",),
        ),
    )(x, W)


# speedup vs baseline: 1.0484x; 1.0484x over previous
import jax
import jax.numpy as jnp
from jax.experimental import pallas as pl
from jax.experimental.pallas import tpu as pltpu

CHANNEL_IN = 256
CHANNEL_OUT = 32768
GROUP = 8
BATCH = 128

COL_BLK = 8192


def _fused_kernel(x_ref, w_ref, o_ref):
    yt = jax.lax.dot_general(
        w_ref[...], x_ref[...], (((0,), (1,)), ((), ())),
        preferred_element_type=jnp.float32)
    y3 = yt.reshape(COL_BLK // GROUP, GROUP, BATCH)
    v = y3
    for k in (1, 2, 4):
        v = jnp.maximum(v, pltpu.roll(v, GROUP - k, 1))
    eqf = (y3 == v).astype(jnp.float32)
    o_ref[...] = eqf.reshape(COL_BLK, BATCH).T
    total = jnp.sum(eqf)

    @pl.when(total > float(COL_BLK // GROUP * BATCH))
    def _exact_tie_break():
        s = jax.lax.broadcasted_iota(
            jnp.int32, (COL_BLK // GROUP, GROUP, BATCH), 1).astype(jnp.float32)
        c = jnp.where(y3 == v, s, jnp.float32(GROUP))
        for k in (1, 2, 4):
            c = jnp.minimum(c, pltpu.roll(c, GROUP - k, 1))
        o_ref[...] = (s == c).astype(jnp.float32).reshape(COL_BLK, BATCH).T


def kernel(x, W):
    grid = (CHANNEL_OUT // COL_BLK,)
    return pl.pallas_call(
        _fused_kernel,
        grid=grid,
        in_specs=[
            pl.BlockSpec((BATCH, CHANNEL_IN), lambda j: (0, 0)),
            pl.BlockSpec((CHANNEL_IN, COL_BLK), lambda j: (0, j)),
        ],
        out_specs=pl.BlockSpec((BATCH, COL_BLK), lambda j: (0, j)),
        out_shape=jax.ShapeDtypeStruct((BATCH, CHANNEL_OUT), jnp.float32),
        compiler_params=pltpu.CompilerParams(
            dimension_semantics=("arbitrary",),
        ),
    )(x, W)
